# Initial kernel scaffold; baseline (speedup 1.0000x reference)
import jax
import jax.numpy as jnp
from jax.experimental import pallas as pl


def kernel(pred_block, pred_line, pred_char, pred_block_logits, pred_line_logits, pred_char_logits, target_sizes):
    raise NotImplementedError("write your pallas kernel here")


# speedup vs baseline: 1.8058x; 1.8058x over previous
import functools

import jax
import jax.numpy as jnp
from jax import lax
from jax.experimental import pallas as pl
from jax.experimental.pallas import tpu as pltpu
from jax.experimental.pallas import tpu_sc as plsc

B, Q, C = 8, 1000, 4096
NW = 32
RPW = (B * Q) // NW
G = 10
NCHUNK = RPW // G
NEG = -3.402823e38


def _body(bb_h, lb_h, ch_h, blg_h, llg_h, cl_h, ts_h, out_h,
          comb, blgv, llgv, tsv, outv, cb0, cb1, sem0, sem1):
    wid = lax.axis_index("s") * 2 + lax.axis_index("c")
    img = wid // (Q // RPW)
    base_row = wid * RPW

    pltpu.sync_copy(bb_h.at[pl.ds(base_row * 4, RPW * 4)], comb.at[pl.ds(0, RPW * 4)])
    pltpu.sync_copy(lb_h.at[pl.ds(base_row * 4, RPW * 4)], comb.at[pl.ds(RPW * 4, RPW * 4)])
    pltpu.sync_copy(ch_h.at[pl.ds(base_row * 16, RPW * 16)], comb.at[pl.ds(RPW * 8, RPW * 16)])
    pltpu.sync_copy(blg_h.at[pl.ds(base_row * 16, RPW * 16)], blgv)
    pltpu.sync_copy(llg_h.at[pl.ds(base_row * 16, RPW * 16)], llgv)
    pltpu.sync_copy(ts_h, tsv.at[pl.ds(0, 16)])

    lanes = lax.iota(jnp.int32, 16)
    lm4 = lanes % 4
    sc1_idx = jnp.where(lanes < 8, 1 - (lanes % 2), lanes % 2) + 2 * img
    sc2_idx = (lanes % 2) + 2 * img
    sc1 = plsc.load_gather(tsv, [sc1_idx])
    sc2 = plsc.load_gather(tsv, [sc2_idx])

    line_off = RPW * 4
    char_off = RPW * 8
    ctr_p = jnp.where(lanes < 8,
                      (lanes % 2) + jnp.where(lanes < 4, 0, line_off),
                      char_off + lanes - 8)
    half_p = jnp.where(lanes < 8, ctr_p + 2, ctr_p)
    stride = jnp.where(lanes < 8, 4, 16)
    coef = jnp.where(lanes < 8, jnp.where(lm4 < 2, -0.5, 0.5), 0.0)

    cbufs = (cb0, cb1)
    sems = (sem0, sem1)

    def chunk_slice(g):
        return cl_h.at[pl.ds((base_row + g * G) * C, G * C)]

    copies = [None] * NCHUNK
    copies[0] = pltpu.async_copy(chunk_slice(0), cb0, sem0)
    for g in range(NCHUNK):
        if g + 1 < NCHUNK:
            copies[g + 1] = pltpu.async_copy(
                chunk_slice(g + 1), cbufs[(g + 1) % 2], sems[(g + 1) % 2])
        copies[g].wait()
        cb = cbufs[g % 2]

        def row_body(rr, _, g=g, cb=cb):
            r = g * G + rr
            cbase = rr * C
            accs = tuple(cb[pl.ds(cbase + k * 16, 16)] for k in range(8))

            def chunk_body(c, a):
                o = cbase + c * 128
                return tuple(jnp.maximum(a[k], cb[pl.ds(o + k * 16, 16)])
                             for k in range(8))

            accs = lax.fori_loop(1, C // 128, chunk_body, accs)
            m = accs[0]
            for k in range(1, 8):
                m = jnp.maximum(m, accs[k])
            first = cb[pl.ds(cbase, 16)]
            v0c = jnp.max(jnp.where(lanes == 0, first, NEG))
            fc = jnp.where(jnp.max(m) > v0c, 1.0, 0.0)

            vb = blgv[pl.ds(16 * r, 16)]
            fb = jnp.where(jnp.max(vb) > jnp.max(jnp.where(lanes == 0, vb, NEG)), 1.0, 0.0)
            vl = llgv[pl.ds(16 * r, 16)]
            fl = jnp.where(jnp.max(vl) > jnp.max(jnp.where(lanes == 0, vl, NEG)), 1.0, 0.0)

            ctr = plsc.load_gather(comb, [ctr_p + r * stride])
            half = plsc.load_gather(comb, [half_p + r * stride])
            mult = jnp.where(lanes < 4, fb, jnp.where(lanes < 8, fl, fc))
            outv[pl.ds(24 * r, 16)] = (ctr + coef * half) * sc1 * mult
            charv = comb[pl.ds(char_off + 16 * r, 16)]
            outv[pl.ds(24 * r + 8, 16)] = charv * sc2 * fc
            return 0

        lax.fori_loop(0, G, row_body, 0)

    pltpu.sync_copy(outv, out_h.at[pl.ds(base_row * 24, RPW * 24)])


@jax.jit
def kernel(pred_block, pred_line, pred_char, pred_block_logits,
           pred_line_logits, pred_char_logits, target_sizes):
    mesh = plsc.VectorSubcoreMesh(core_axis_name="c", subcore_axis_name="s")
    run = functools.partial(
        pl.kernel,
        mesh=mesh,
        compiler_params=pltpu.CompilerParams(needs_layout_passes=False),
        out_type=jax.ShapeDtypeStruct((B * Q * 24,), jnp.float32),
        scratch_types=[
            pltpu.VMEM((RPW * 24,), jnp.float32),
            pltpu.VMEM((RPW * 16,), jnp.float32),
            pltpu.VMEM((RPW * 16,), jnp.float32),
            pltpu.VMEM((128,), jnp.float32),
            pltpu.VMEM((RPW * 24,), jnp.float32),
            pltpu.VMEM((G * C,), jnp.float32),
            pltpu.VMEM((G * C,), jnp.float32),
            pltpu.SemaphoreType.DMA,
            pltpu.SemaphoreType.DMA,
        ],
    )(_body)
    out = run(pred_block.reshape(-1), pred_line.reshape(-1),
              pred_char.reshape(-1), pred_block_logits.reshape(-1),
              pred_line_logits.reshape(-1), pred_char_logits.reshape(-1),
              target_sizes.reshape(-1))
    return out.reshape(B, Q, 24)


# speedup vs baseline: 2.0466x; 1.1333x over previous
import functools

import jax
import jax.numpy as jnp
from jax import lax
from jax.experimental import pallas as pl
from jax.experimental.pallas import tpu as pltpu
from jax.experimental.pallas import tpu_sc as plsc

B, Q, C = 8, 1000, 4096
NW = 32
RPW = (B * Q) // NW
W1 = 128
NG = (RPW + 15) // 16
NEG = -3.402823e38


FB, FL, FC = 0, NG * 16, 2 * NG * 16


def _body(bb_h, lb_h, ch_h, blg_h, llg_h, cl_h, ts_h, out_h,
          comb, blgv, llgv, tsv, outv, buf2, rowbuf, flags, sem):
    wid = lax.axis_index("s") * 2 + lax.axis_index("c")
    img = wid // (Q // RPW)
    base_row = wid * RPW

    cp = pltpu.async_copy(
        cl_h.at[pl.ds(base_row, RPW), pl.ds(0, W1)], buf2, sem)
    pltpu.sync_copy(bb_h.at[pl.ds(base_row * 4, RPW * 4)], comb.at[pl.ds(0, RPW * 4)])
    pltpu.sync_copy(lb_h.at[pl.ds(base_row * 4, RPW * 4)], comb.at[pl.ds(RPW * 4, RPW * 4)])
    pltpu.sync_copy(ch_h.at[pl.ds(base_row * 16, RPW * 16)], comb.at[pl.ds(RPW * 8, RPW * 16)])
    pltpu.sync_copy(blg_h.at[pl.ds(base_row * 16, RPW * 16)], blgv)
    pltpu.sync_copy(llg_h.at[pl.ds(base_row * 16, RPW * 16)], llgv)
    pltpu.sync_copy(ts_h, tsv.at[pl.ds(0, 16)])
    cp.wait()

    lanes = lax.iota(jnp.int32, 16)

    def char_group(gi, _):
        rows = jnp.minimum(gi * 16 + lanes, RPW - 1)
        v0 = plsc.load_gather(buf2, [rows, lanes * 0])

        def col(c, acc):
            v = plsc.load_gather(buf2, [rows, lanes * 0 + c])
            return jnp.logical_or(acc, v > v0)

        acc = lax.fori_loop(1, W1, col, v0 != v0)
        flags[pl.ds(FC + gi * 16, 16)] = jnp.where(acc, 1.0, -1.0)
        return 0

    lax.fori_loop(0, NG, char_group, 0)

    def bl_group(gi, _):
        rows = jnp.minimum(gi * 16 + lanes, RPW - 1)
        base = rows * 16
        vb0 = plsc.load_gather(blgv, [base])
        vl0 = plsc.load_gather(llgv, [base])

        def col(c, accs):
            ab, al = accs
            ab = jnp.logical_or(ab, plsc.load_gather(blgv, [base + c]) > vb0)
            al = jnp.logical_or(al, plsc.load_gather(llgv, [base + c]) > vl0)
            return ab, al

        ab, al = lax.fori_loop(1, 16, col, (vb0 != vb0, vl0 != vl0))
        flags[pl.ds(FB + gi * 16, 16)] = jnp.where(ab, 1.0, 0.0)
        flags[pl.ds(FL + gi * 16, 16)] = jnp.where(al, 1.0, 0.0)
        return 0

    lax.fori_loop(0, NG, bl_group, 0)

    def resolve(r, _):
        fc_here = plsc.load_gather(flags, [FC + r + lanes * 0])

        @pl.when(fc_here[0] < 0.0)
        def _():
            pltpu.sync_copy(cl_h.at[pl.ds(base_row + r, 1), :], rowbuf)
            accs = tuple(rowbuf[0, pl.ds(k * 16, 16)] for k in range(8))

            def chunk(c, a):
                o = c * 128
                return tuple(jnp.maximum(a[k], rowbuf[0, pl.ds(o + k * 16, 16)])
                             for k in range(8))

            accs = lax.fori_loop(1, C // 128, chunk, accs)
            m = accs[0]
            for k in range(1, 8):
                m = jnp.maximum(m, accs[k])
            first = rowbuf[0, pl.ds(0, 16)]
            v0 = jnp.max(jnp.where(lanes == 0, first, NEG))
            val = jnp.where(jnp.max(m) > v0, 1.0, 0.0) + lanes * 0.0
            plsc.store_scatter(flags, [FC + r + lanes * 0], val, mask=lanes == 0)
        return 0

    lax.fori_loop(0, RPW, resolve, 0)

    line_off = RPW * 4
    char_off = RPW * 8
    lm2 = lanes % 2
    ctr_p = jnp.where(lanes < 8,
                      lm2 + jnp.where(lanes < 4, 0, line_off),
                      char_off + lanes - 8)
    half_p = jnp.where(lanes < 8, ctr_p + 2, ctr_p)
    stride = jnp.where(lanes < 8, 4, 16)
    coef = jnp.where(lanes < 8, jnp.where(lanes % 4 < 2, -0.5, 0.5), 0.0)
    sc1 = plsc.load_gather(tsv, [jnp.where(lanes < 8, 1 - lm2, lm2) + 2 * img])
    sc2 = plsc.load_gather(tsv, [lm2 + 2 * img])
    flag_off = jnp.where(lanes < 4, FB, jnp.where(lanes < 8, FL, FC))

    def emit(r, _):
        ctr = plsc.load_gather(comb, [ctr_p + r * stride])
        half = plsc.load_gather(comb, [half_p + r * stride])
        mult = plsc.load_gather(flags, [flag_off + r])
        fc = plsc.load_gather(flags, [FC + r + lanes * 0])
        outv[pl.ds(24 * r, 16)] = (ctr + coef * half) * sc1 * mult
        charv = comb[pl.ds(char_off + 16 * r, 16)]
        outv[pl.ds(24 * r + 8, 16)] = charv * sc2 * fc
        return 0

    lax.fori_loop(0, RPW, emit, 0)

    pltpu.sync_copy(outv, out_h.at[pl.ds(base_row * 24, RPW * 24)])


@jax.jit
def kernel(pred_block, pred_line, pred_char, pred_block_logits,
           pred_line_logits, pred_char_logits, target_sizes):
    mesh = plsc.VectorSubcoreMesh(core_axis_name="c", subcore_axis_name="s")
    run = functools.partial(
        pl.kernel,
        mesh=mesh,
        compiler_params=pltpu.CompilerParams(
            needs_layout_passes=False, use_tc_tiling_on_sc=False),
        out_type=jax.ShapeDtypeStruct((B * Q * 24,), jnp.float32),
        scratch_types=[
            pltpu.VMEM((RPW * 24,), jnp.float32),
            pltpu.VMEM((RPW * 16,), jnp.float32),
            pltpu.VMEM((RPW * 16,), jnp.float32),
            pltpu.VMEM((128,), jnp.float32),
            pltpu.VMEM((RPW * 24,), jnp.float32),
            pltpu.VMEM((RPW, W1), jnp.float32),
            pltpu.VMEM((1, C), jnp.float32),
            pltpu.VMEM((3 * NG * 16,), jnp.float32),
            pltpu.SemaphoreType.DMA,
        ],
    )(_body)
    out = run(pred_block.reshape(-1), pred_line.reshape(-1),
              pred_char.reshape(-1), pred_block_logits.reshape(-1),
              pred_line_logits.reshape(-1),
              pred_char_logits.reshape(B * Q, C),
              target_sizes.reshape(-1))
    return out.reshape(B, Q, 24)


# speedup vs baseline: 3.6722x; 1.7943x over previous
import functools

import jax
import jax.numpy as jnp
from jax import lax
from jax.experimental import pallas as pl
from jax.experimental.pallas import tpu as pltpu
from jax.experimental.pallas import tpu_sc as plsc

B, Q, C = 8, 1000, 4096
QPW = 256
QSTEP = 248
NG = QPW // 16
NEG = -3.402823e38
FB, FL, FC = 0, QPW, 2 * QPW


def _body(bb_h, lb_h, ch_h, blg_h, llg_h, cl_h, ts_h, out_h,
          comb, blgv, llgv, tsv, outv, buf2, rowbuf, flags, sem):
    wid = lax.axis_index("s") * 2 + lax.axis_index("c")
    img = wid // 4
    qs = (wid % 4) * QSTEP
    base_row = img * Q + qs

    cp = pltpu.async_copy(cl_h.at[img, pl.ds(qs // 8, QPW // 8), 0], buf2, sem)
    pltpu.sync_copy(bb_h.at[pl.ds(base_row * 4, QPW * 4)], comb.at[pl.ds(0, QPW * 4)])
    pltpu.sync_copy(lb_h.at[pl.ds(base_row * 4, QPW * 4)], comb.at[pl.ds(QPW * 4, QPW * 4)])
    pltpu.sync_copy(ch_h.at[pl.ds(base_row * 16, QPW * 16)], comb.at[pl.ds(QPW * 8, QPW * 16)])
    pltpu.sync_copy(blg_h.at[pl.ds(base_row * 16, QPW * 16)], blgv)
    pltpu.sync_copy(llg_h.at[pl.ds(base_row * 16, QPW * 16)], llgv)
    pltpu.sync_copy(ts_h, tsv.at[pl.ds(0, 16)])
    cp.wait()

    lanes = lax.iota(jnp.int32, 16)

    def char_group(gi, _):
        rows = gi * 16 + lanes
        tq = rows // 8
        qi = rows % 8
        v0 = plsc.load_gather(buf2, [tq, qi, lanes * 0])

        def col(c, acc):
            v = plsc.load_gather(buf2, [tq, qi, lanes * 0 + c])
            return jnp.logical_or(acc, v > v0)

        acc = lax.fori_loop(1, 128, col, v0 != v0)
        flags[pl.ds(FC + gi * 16, 16)] = jnp.where(acc, 1.0, -1.0)
        return 0

    lax.fori_loop(0, NG, char_group, 0)

    def bl_group(gi, _):
        base = (gi * 16 + lanes) * 16
        vb0 = plsc.load_gather(blgv, [base])
        vl0 = plsc.load_gather(llgv, [base])

        def col(c, accs):
            ab, al = accs
            ab = jnp.logical_or(ab, plsc.load_gather(blgv, [base + c]) > vb0)
            al = jnp.logical_or(al, plsc.load_gather(llgv, [base + c]) > vl0)
            return ab, al

        ab, al = lax.fori_loop(1, 16, col, (vb0 != vb0, vl0 != vl0))
        flags[pl.ds(FB + gi * 16, 16)] = jnp.where(ab, 1.0, 0.0)
        flags[pl.ds(FL + gi * 16, 16)] = jnp.where(al, 1.0, 0.0)
        return 0

    lax.fori_loop(0, NG, bl_group, 0)

    def resolve(r, _):
        fc_here = plsc.load_gather(flags, [FC + r + lanes * 0])

        @pl.when(fc_here[0] < 0.0)
        def _():
            pltpu.sync_copy(
                cl_h.at[img, qs // 8 + r // 8, pl.ds(1, 31), r % 8], rowbuf)
            accs = tuple(rowbuf[0, pl.ds(k * 16, 16)] for k in range(8))

            def chunk(t, a):
                return tuple(jnp.maximum(a[k], rowbuf[t, pl.ds(k * 16, 16)])
                             for k in range(8))

            accs = lax.fori_loop(1, 31, chunk, accs)
            m = accs[0]
            for k in range(1, 8):
                m = jnp.maximum(m, accs[k])
            v0v = plsc.load_gather(
                buf2, [lanes * 0 + r // 8, lanes * 0 + r % 8, lanes * 0])
            val = jnp.where(jnp.max(m) > v0v[0], 1.0, 0.0) + lanes * 0.0
            plsc.store_scatter(flags, [FC + r + lanes * 0], val, mask=lanes == 0)
        return 0

    lax.fori_loop(0, QPW, resolve, 0)

    line_off = QPW * 4
    char_off = QPW * 8
    lm2 = lanes % 2
    ctr_p = jnp.where(lanes < 8,
                      lm2 + jnp.where(lanes < 4, 0, line_off),
                      char_off + lanes - 8)
    half_p = jnp.where(lanes < 8, ctr_p + 2, ctr_p)
    stride = jnp.where(lanes < 8, 4, 16)
    coef = jnp.where(lanes < 8, jnp.where(lanes % 4 < 2, -0.5, 0.5), 0.0)
    sc1 = plsc.load_gather(tsv, [jnp.where(lanes < 8, 1 - lm2, lm2) + 2 * img])
    sc2 = plsc.load_gather(tsv, [lm2 + 2 * img])
    flag_off = jnp.where(lanes < 4, FB, jnp.where(lanes < 8, FL, FC))

    def emit(r, _):
        ctr = plsc.load_gather(comb, [ctr_p + r * stride])
        half = plsc.load_gather(comb, [half_p + r * stride])
        mult = plsc.load_gather(flags, [flag_off + r])
        fc = plsc.load_gather(flags, [FC + r + lanes * 0])
        outv[pl.ds(24 * r, 16)] = (ctr + coef * half) * sc1 * mult
        charv = comb[pl.ds(char_off + 16 * r, 16)]
        outv[pl.ds(24 * r + 8, 16)] = charv * sc2 * fc
        return 0

    lax.fori_loop(0, QPW, emit, 0)

    pltpu.sync_copy(outv, out_h.at[pl.ds(base_row * 24, QPW * 24)])


@jax.jit
def kernel(pred_block, pred_line, pred_char, pred_block_logits,
           pred_line_logits, pred_char_logits, target_sizes):
    mesh = plsc.VectorSubcoreMesh(core_axis_name="c", subcore_axis_name="s")
    run = functools.partial(
        pl.kernel,
        mesh=mesh,
        compiler_params=pltpu.CompilerParams(
            needs_layout_passes=False, use_tc_tiling_on_sc=False),
        out_type=jax.ShapeDtypeStruct((B * Q * 24,), jnp.float32),
        scratch_types=[
            pltpu.VMEM((QPW * 24,), jnp.float32),
            pltpu.VMEM((QPW * 16,), jnp.float32),
            pltpu.VMEM((QPW * 16,), jnp.float32),
            pltpu.VMEM((128,), jnp.float32),
            pltpu.VMEM((QPW * 24,), jnp.float32),
            pltpu.VMEM((QPW // 8, 8, 128), jnp.float32),
            pltpu.VMEM((31, 128), jnp.float32),
            pltpu.VMEM((3 * QPW,), jnp.float32),
            pltpu.SemaphoreType.DMA,
        ],
    )(_body)
    out = run(pred_block.reshape(-1), pred_line.reshape(-1),
              pred_char.reshape(-1), pred_block_logits.reshape(-1),
              pred_line_logits.reshape(-1),
              pred_char_logits.reshape(B, Q // 8, 8, C // 128, 128)
                              .transpose(0, 1, 3, 2, 4),
              target_sizes.reshape(-1))
    return out.reshape(B, Q, 24)


# speedup vs baseline: 6.0862x; 1.6574x over previous
import functools

import jax
import jax.numpy as jnp
from jax import lax
from jax.experimental import pallas as pl
from jax.experimental.pallas import tpu as pltpu
from jax.experimental.pallas import tpu_sc as plsc

B, Q, C = 8, 1000, 4096
QPW = 256
QSTEP = 248
NG = QPW // 16
NEG = -3.402823e38
FB, FL, FC = 0, QPW, 2 * QPW


def _body(bb_h, lb_h, ch_h, blg_h, llg_h, cl_h, ts_h, out_h,
          comb, blgv, llgv, tsv, outv, buf2, rowbuf, flags, sem):
    wid = lax.axis_index("s") * 2 + lax.axis_index("c")
    img = wid // 4
    qs = (wid % 4) * QSTEP
    q_sl = pl.ds(qs, QPW)

    cps = [
        pltpu.async_copy(cl_h.at[img, pl.ds(qs // 8, QPW // 8), 0], buf2, sem),
        pltpu.async_copy(bb_h.at[img, :, q_sl], comb.at[pl.ds(0, 4)], sem),
        pltpu.async_copy(lb_h.at[img, :, q_sl], comb.at[pl.ds(4, 4)], sem),
        pltpu.async_copy(ch_h.at[img, :, q_sl], comb.at[pl.ds(8, 16)], sem),
        pltpu.async_copy(blg_h.at[img, :, q_sl], blgv, sem),
        pltpu.async_copy(llg_h.at[img, :, q_sl], llgv, sem),
        pltpu.async_copy(ts_h, tsv.at[:, pl.ds(0, 8)], sem),
    ]
    for cp in cps:
        cp.wait()

    lanes = lax.iota(jnp.int32, 16)

    def char_group(gi, _):
        rows = gi * 16 + lanes
        tq = rows // 8
        qi = rows % 8
        v0 = plsc.load_gather(buf2, [tq, qi, lanes * 0])
        acc = v0 != v0
        for c in range(1, 128):
            acc = jnp.logical_or(
                acc, plsc.load_gather(buf2, [tq, qi, lanes * 0 + c]) > v0)
        flags[pl.ds(FC + gi * 16, 16)] = jnp.where(acc, 1.0, -1.0)
        return 0

    lax.fori_loop(0, NG, char_group, 0)

    def bl_group(gi, _):
        g_sl = pl.ds(gi * 16, 16)
        vb0 = blgv[0, g_sl]
        vl0 = llgv[0, g_sl]
        ab = vb0 != vb0
        al = ab
        for c in range(1, 16):
            ab = jnp.logical_or(ab, blgv[c, g_sl] > vb0)
            al = jnp.logical_or(al, llgv[c, g_sl] > vl0)
        flags[pl.ds(FB + gi * 16, 16)] = jnp.where(ab, 1.0, 0.0)
        flags[pl.ds(FL + gi * 16, 16)] = jnp.where(al, 1.0, 0.0)
        return 0

    lax.fori_loop(0, NG, bl_group, 0)

    def resolve_group(gi, _):
        fvec = flags[pl.ds(FC + gi * 16, 16)]

        @pl.when(jnp.min(fvec) < 0.0)
        def _():
            def resolve(rr, _):
                r = gi * 16 + rr
                fc_here = plsc.load_gather(flags, [FC + r + lanes * 0])

                @pl.when(fc_here[0] < 0.0)
                def _():
                    pltpu.sync_copy(
                        cl_h.at[img, qs // 8 + r // 8, pl.ds(1, 31), r % 8],
                        rowbuf)
                    accs = tuple(rowbuf[0, pl.ds(k * 16, 16)] for k in range(8))

                    def chunk(t, a):
                        return tuple(
                            jnp.maximum(a[k], rowbuf[t, pl.ds(k * 16, 16)])
                            for k in range(8))

                    accs = lax.fori_loop(1, 31, chunk, accs)
                    m = accs[0]
                    for k in range(1, 8):
                        m = jnp.maximum(m, accs[k])
                    v0v = plsc.load_gather(
                        buf2, [lanes * 0 + r // 8, lanes * 0 + r % 8, lanes * 0])
                    val = jnp.where(jnp.max(m) > v0v[0], 1.0, 0.0) + lanes * 0.0
                    plsc.store_scatter(flags, [FC + r + lanes * 0], val,
                                       mask=lanes == 0)
                return 0

            lax.fori_loop(0, 16, resolve, 0)
        return 0

    lax.fori_loop(0, NG, resolve_group, 0)

    hsp = plsc.load_gather(tsv, [lanes * 0, lanes * 0 + img])
    wsp = plsc.load_gather(tsv, [lanes * 0 + 1, lanes * 0 + img])

    def emit(gi, _):
        g_sl = pl.ds(gi * 16, 16)
        fb = flags[pl.ds(FB + gi * 16, 16)]
        fl = flags[pl.ds(FL + gi * 16, 16)]
        fc = flags[pl.ds(FC + gi * 16, 16)]
        for base, f in ((0, fb), (4, fl)):
            cx = comb[base + 0, g_sl]
            cy = comb[base + 1, g_sl]
            hw = comb[base + 2, g_sl] * 0.5
            hh = comb[base + 3, g_sl] * 0.5
            outv[base + 0, g_sl] = (cx - hw) * wsp * f
            outv[base + 1, g_sl] = (cy - hh) * hsp * f
            outv[base + 2, g_sl] = (cx + hw) * wsp * f
            outv[base + 3, g_sl] = (cy + hh) * hsp * f
        for c in range(16):
            sc = hsp if c % 2 == 0 else wsp
            outv[8 + c, g_sl] = comb[8 + c, g_sl] * sc * fc
        return 0

    lax.fori_loop(0, NG, emit, 0)

    pltpu.sync_copy(outv, out_h.at[img, :, q_sl])


@jax.jit
def kernel(pred_block, pred_line, pred_char, pred_block_logits,
           pred_line_logits, pred_char_logits, target_sizes):
    mesh = plsc.VectorSubcoreMesh(core_axis_name="c", subcore_axis_name="s")
    run = functools.partial(
        pl.kernel,
        mesh=mesh,
        compiler_params=pltpu.CompilerParams(
            needs_layout_passes=False, use_tc_tiling_on_sc=False),
        out_type=jax.ShapeDtypeStruct((B, 24, Q), jnp.float32),
        scratch_types=[
            pltpu.VMEM((24, QPW), jnp.float32),
            pltpu.VMEM((16, QPW), jnp.float32),
            pltpu.VMEM((16, QPW), jnp.float32),
            pltpu.VMEM((2, 128), jnp.float32),
            pltpu.VMEM((24, QPW), jnp.float32),
            pltpu.VMEM((QPW // 8, 8, 128), jnp.float32),
            pltpu.VMEM((31, 128), jnp.float32),
            pltpu.VMEM((3 * QPW,), jnp.float32),
            pltpu.SemaphoreType.DMA,
        ],
    )(_body)
    out = run(pred_block.transpose(0, 2, 1), pred_line.transpose(0, 2, 1),
              pred_char.transpose(0, 2, 1),
              pred_block_logits.transpose(0, 2, 1),
              pred_line_logits.transpose(0, 2, 1),
              pred_char_logits.reshape(B, Q // 8, 8, C // 128, 128)
                              .transpose(0, 1, 3, 2, 4),
              target_sizes.transpose(1, 0))
    return out.transpose(0, 2, 1)
